# baseline (speedup 1.0000x reference)
import functools

import jax
import jax.numpy as jnp
from jax import lax
from jax.experimental import pallas as pl
from jax.experimental.pallas import tpu as pltpu
from jax.experimental.pallas import tpu_sc as plsc

N_NODES = 10000
IN_DIM = 128
HID = 64
N_EDGES = 320000

NC = 2
NS = 16
NW = NC * NS
EPW = N_EDGES // NW
CHUNK = 128
NFULL = EPW // CHUNK
TAIL = EPW - NFULL * CHUNK
N_PAD = 10240
RPT = N_PAD // NS
NBUF = 4
DW = N_PAD // 128

_mesh = plsc.VectorSubcoreMesh(
    core_axis_name="c", subcore_axis_name="s", num_cores=NC, num_subcores=NS
)


def _chunk(idx_v, j):
    base = pl.multiple_of(j * CHUNK, CHUNK)
    return idx_v.at[pl.ds(base, CHUNK)]


def _tail(idx_v):
    return idx_v.at[pl.ds(NFULL * CHUNK, TAIL)]


@functools.partial(
    pl.kernel,
    out_type=jax.ShapeDtypeStruct((NC, N_PAD), jnp.float32),
    mesh=_mesh,
    scratch_types=[
        pltpu.VMEM((EPW,), jnp.int32),
        pltpu.VMEM((CHUNK,), jnp.float32),
        pltpu.VMEM((RPT,), jnp.float32),
        pltpu.VMEM_SHARED((N_PAD,), jnp.float32),
        pltpu.SemaphoreType.DMA,
        pltpu.SemaphoreType.DMA,
    ],
    compiler_params=pltpu.CompilerParams(use_tc_tiling_on_sc=False),
)
def _deg_kernel(dst_hbm, out_hbm, idx_v, ones_v, zrow_v, acc, sem_a, sem_b):
    c = lax.axis_index("c")
    s = lax.axis_index("s")
    wid = s * NC + c
    pltpu.sync_copy(dst_hbm.at[pl.ds(wid * EPW, EPW)], idx_v)
    one16 = jnp.ones((16,), jnp.float32)
    zero16 = jnp.zeros((16,), jnp.float32)
    for i in range(CHUNK // 16):
        ones_v[pl.ds(i * 16, 16)] = one16

    def zbody(i, carry):
        zrow_v[pl.ds(i * 16, 16)] = zero16
        return carry

    lax.fori_loop(0, RPT // 16, zbody, 0)
    pltpu.sync_copy(zrow_v, acc.at[pl.ds(s * RPT, RPT)])
    plsc.subcore_barrier()

    def fire(j, sem):
        return pltpu.async_copy(ones_v, acc.at[_chunk(idx_v, j)], sem,
                                add=True)

    def drain(j, sem):
        pltpu.make_async_copy(ones_v, acc.at[_chunk(idx_v, j)], sem).wait()

    fire(0, sem_a)

    def body(k, carry):
        j0 = 2 * k
        fire(j0 + 1, sem_b)
        drain(j0, sem_a)

        @pl.when(j0 + 2 < NFULL)
        def _():
            fire(j0 + 2, sem_a)

        drain(j0 + 1, sem_b)
        return carry

    lax.fori_loop(0, NFULL // 2, body, 0)
    pltpu.sync_copy(ones_v.at[pl.ds(0, TAIL)], acc.at[_tail(idx_v)],
                    add=True)
    plsc.subcore_barrier()

    @pl.when(s == 0)
    def _():
        pltpu.sync_copy(acc, out_hbm.at[c])


@functools.partial(
    pl.kernel,
    out_type=jax.ShapeDtypeStruct((N_PAD, 2 * HID), jnp.float32),
    mesh=_mesh,
    scratch_types=[
        pltpu.VMEM((EPW,), jnp.int32),
        pltpu.VMEM((EPW,), jnp.int32),
        [pltpu.VMEM((CHUNK, HID), jnp.float32) for _ in range(NBUF)],
        pltpu.VMEM_SHARED((N_PAD, HID), jnp.float32),
        [pltpu.SemaphoreType.DMA for _ in range(NBUF)],
        [pltpu.SemaphoreType.DMA for _ in range(NBUF)],
    ],
    compiler_params=pltpu.CompilerParams(use_tc_tiling_on_sc=False),
)
def _agg_kernel(h_hbm, src_hbm, dst_hbm, out_hbm,
                src_v, dst_v, bufs, acc, gsems, ssems):
    c = lax.axis_index("c")
    s = lax.axis_index("s")
    wid = s * NC + c
    pltpu.sync_copy(src_hbm.at[pl.ds(wid * EPW, EPW)], src_v)
    pltpu.sync_copy(dst_hbm.at[pl.ds(wid * EPW, EPW)], dst_v)

    zero16 = jnp.zeros((16,), jnp.float32)

    def zbody(r, carry):
        for cc in range(HID // 16):
            bufs[0][r, pl.ds(cc * 16, 16)] = zero16
        return carry

    lax.fori_loop(0, CHUNK, zbody, 0)
    r0 = s * RPT
    for i in range(RPT // CHUNK):
        pltpu.sync_copy(bufs[0], acc.at[pl.ds(r0 + i * CHUNK, CHUNK)])
    plsc.subcore_barrier()

    def fire_gather(i, j):
        pltpu.async_copy(h_hbm.at[_chunk(src_v, j)], bufs[i], gsems[i])

    def wait_gather(i, j):
        pltpu.make_async_copy(h_hbm.at[_chunk(src_v, j)], bufs[i],
                              gsems[i]).wait()

    for i in range(NBUF):
        fire_gather(i, i)

    def body(k, carry):
        j0 = NBUF * k
        descs = []
        for i in range(NBUF):
            wait_gather(i, j0 + i)
            descs.append(
                pltpu.async_copy(bufs[i], acc.at[_chunk(dst_v, j0 + i)],
                                 ssems[i], add=True))
        for i in range(NBUF):
            descs[i].wait()
            jn = j0 + NBUF + i

            @pl.when(jn < NFULL)
            def _():
                fire_gather(i, jn)

        return carry

    lax.fori_loop(0, NFULL // NBUF, body, 0)
    pltpu.async_copy(h_hbm.at[_tail(src_v)], bufs[2].at[pl.ds(0, TAIL)],
                     gsems[2])
    d = []
    for i in range(NFULL - NBUF * (NFULL // NBUF)):
        j = NBUF * (NFULL // NBUF) + i
        wait_gather(i, j)
        d.append(pltpu.async_copy(bufs[i], acc.at[_chunk(dst_v, j)],
                                  ssems[i], add=True))
    pltpu.make_async_copy(h_hbm.at[_tail(src_v)], bufs[2].at[pl.ds(0, TAIL)],
                          gsems[2]).wait()
    d.append(pltpu.async_copy(bufs[2].at[pl.ds(0, TAIL)],
                              acc.at[_tail(dst_v)], ssems[2], add=True))
    for desc in d:
        desc.wait()

    plsc.subcore_barrier()

    @pl.when(c == 0)
    def _():
        pltpu.sync_copy(acc.at[pl.ds(r0, RPT)],
                        out_hbm.at[pl.ds(r0, RPT), pl.ds(0, HID)])

    @pl.when(c == 1)
    def _():
        pltpu.sync_copy(acc.at[pl.ds(r0, RPT)],
                        out_hbm.at[pl.ds(r0, RPT), pl.ds(HID, HID)])


def _tc_mm_body(x_ref, w_ref, mm_ref):
    mm_ref[...] = jnp.dot(x_ref[...], w_ref[...],
                          preferred_element_type=jnp.float32)


def _dis_matrix(disw):
    d3 = lax.broadcast_in_dim(disw, (DW, HID, 128), (0, 2))
    return jnp.swapaxes(d3, 1, 2).reshape(N_PAD, HID)


def _tc1_body(mm_ref, d0_ref, d1_ref, dis_ref, hs_ref):
    disw = lax.rsqrt(1.0 + d0_ref[...] + d1_ref[...])
    dis_ref[...] = disw
    dmat = _dis_matrix(disw)
    hs_ref[pl.ds(0, N_NODES), :] = mm_ref[...] * dmat[:N_NODES, :]
    hs_ref[pl.ds(N_NODES, N_PAD - N_NODES), :] = jnp.zeros(
        (N_PAD - N_NODES, HID), jnp.float32)


def _tc2_body(p_ref, hs1_ref, dis_ref, b1_ref, w2_ref, hs2_ref):
    dmat = _dis_matrix(dis_ref[...])
    p = p_ref[...]
    h1 = jnp.maximum(
        dmat * (p[:, :HID] + p[:, HID:] + hs1_ref[...]) + b1_ref[...], 0.0
    )
    hs2_ref[...] = (
        jnp.dot(h1, w2_ref[...], preferred_element_type=jnp.float32) * dmat
    )


def _tc3_body(q_ref, hs2_ref, dis_ref, b2_ref, out_ref):
    dmat = _dis_matrix(dis_ref[...])
    q = q_ref[...]
    full = (
        dmat * (q[:, :HID] + q[:, HID:] + hs2_ref[...])
        + b2_ref[...]
    )
    out_ref[...] = full[:N_NODES, :]


def kernel(x, edge_index, W1, b1, W2, b2):
    x = x.astype(jnp.float32)
    ei = edge_index.astype(jnp.int32)
    dst1 = ei[1]
    ei2, _ = lax.optimization_barrier((ei, dst1))
    src1 = ei2[0]

    degp = _deg_kernel(dst1)
    d0 = degp[0].reshape(DW, 128)
    d1 = degp[1].reshape(DW, 128)

    mm = pl.pallas_call(
        _tc_mm_body,
        out_shape=jax.ShapeDtypeStruct((N_NODES, HID), jnp.float32),
    )(x, W1)

    dis, hs1 = pl.pallas_call(
        _tc1_body,
        out_shape=[
            jax.ShapeDtypeStruct((DW, 128), jnp.float32),
            jax.ShapeDtypeStruct((N_PAD, HID), jnp.float32),
        ],
    )(mm, d0, d1)

    p = _agg_kernel(hs1, src1, dst1)

    hs2 = pl.pallas_call(
        _tc2_body,
        out_shape=jax.ShapeDtypeStruct((N_PAD, HID), jnp.float32),
    )(p, hs1, dis, b1.reshape(1, HID), W2)

    q = _agg_kernel(hs2, src1, dst1)

    out = pl.pallas_call(
        _tc3_body,
        out_shape=jax.ShapeDtypeStruct((N_NODES, HID), jnp.float32),
    )(q, hs2, dis, b2.reshape(1, HID))
    return out


# speedup vs baseline: 1.0519x; 1.0519x over previous
import functools

import jax
import jax.numpy as jnp
from jax import lax
from jax.experimental import pallas as pl
from jax.experimental.pallas import tpu as pltpu
from jax.experimental.pallas import tpu_sc as plsc

N_NODES = 10000
IN_DIM = 128
HID = 64
N_EDGES = 320000

NC = 2
NS = 16
NW = NC * NS
EPW = N_EDGES // NW
CHUNK = 128
NFULL = EPW // CHUNK
TAIL = EPW - NFULL * CHUNK
N_PAD = 10240
RPT = N_PAD // NS
NBUF = 4
DW = N_PAD // 128

_mesh = plsc.VectorSubcoreMesh(
    core_axis_name="c", subcore_axis_name="s", num_cores=NC, num_subcores=NS
)


def _chunk(idx_v, j):
    base = pl.multiple_of(j * CHUNK, CHUNK)
    return idx_v.at[pl.ds(base, CHUNK)]


def _tail(idx_v):
    return idx_v.at[pl.ds(NFULL * CHUNK, TAIL)]


@functools.partial(
    pl.kernel,
    out_type=jax.ShapeDtypeStruct((NC, N_PAD), jnp.float32),
    mesh=_mesh,
    scratch_types=[
        pltpu.VMEM((EPW,), jnp.int32),
        pltpu.VMEM((CHUNK,), jnp.float32),
        pltpu.VMEM((RPT,), jnp.float32),
        pltpu.VMEM_SHARED((N_PAD,), jnp.float32),
        pltpu.SemaphoreType.DMA,
        pltpu.SemaphoreType.DMA,
    ],
    compiler_params=pltpu.CompilerParams(use_tc_tiling_on_sc=False),
)
def _deg_kernel(dst_hbm, out_hbm, idx_v, ones_v, zrow_v, acc, sem_a, sem_b):
    c = lax.axis_index("c")
    s = lax.axis_index("s")
    wid = s * NC + c
    pltpu.sync_copy(dst_hbm.at[pl.ds(wid * EPW, EPW)], idx_v)
    one16 = jnp.ones((16,), jnp.float32)
    zero16 = jnp.zeros((16,), jnp.float32)
    for i in range(CHUNK // 16):
        ones_v[pl.ds(i * 16, 16)] = one16

    def zbody(i, carry):
        zrow_v[pl.ds(i * 16, 16)] = zero16
        return carry

    lax.fori_loop(0, RPT // 16, zbody, 0)
    pltpu.sync_copy(zrow_v, acc.at[pl.ds(s * RPT, RPT)])
    plsc.subcore_barrier()

    def fire(j, sem):
        return pltpu.async_copy(ones_v, acc.at[_chunk(idx_v, j)], sem,
                                add=True)

    def drain(j, sem):
        pltpu.make_async_copy(ones_v, acc.at[_chunk(idx_v, j)], sem).wait()

    fire(0, sem_a)

    def body(k, carry):
        j0 = 2 * k
        fire(j0 + 1, sem_b)
        drain(j0, sem_a)

        @pl.when(j0 + 2 < NFULL)
        def _():
            fire(j0 + 2, sem_a)

        drain(j0 + 1, sem_b)
        return carry

    lax.fori_loop(0, NFULL // 2, body, 0)
    pltpu.sync_copy(ones_v.at[pl.ds(0, TAIL)], acc.at[_tail(idx_v)],
                    add=True)
    plsc.subcore_barrier()

    @pl.when(s == 0)
    def _():
        pltpu.sync_copy(acc, out_hbm.at[c])


@functools.partial(
    pl.kernel,
    out_type=jax.ShapeDtypeStruct((N_PAD, 2 * HID), jnp.float32),
    mesh=_mesh,
    scratch_types=[
        pltpu.VMEM((EPW,), jnp.int32),
        pltpu.VMEM((EPW,), jnp.int32),
        [pltpu.VMEM((CHUNK, HID), jnp.float32) for _ in range(NBUF)],
        pltpu.VMEM_SHARED((N_PAD, HID), jnp.float32),
        [pltpu.SemaphoreType.DMA for _ in range(NBUF)],
        [pltpu.SemaphoreType.DMA for _ in range(NBUF)],
    ],
    compiler_params=pltpu.CompilerParams(use_tc_tiling_on_sc=False),
)
def _agg_kernel(h_hbm, src_hbm, dst_hbm, out_hbm,
                src_v, dst_v, bufs, acc, gsems, ssems):
    c = lax.axis_index("c")
    s = lax.axis_index("s")
    wid = s * NC + c
    pltpu.sync_copy(src_hbm.at[pl.ds(wid * EPW, EPW)], src_v)
    pltpu.sync_copy(dst_hbm.at[pl.ds(wid * EPW, EPW)], dst_v)

    zero16 = jnp.zeros((16,), jnp.float32)

    def zbody(r, carry):
        for cc in range(HID // 16):
            bufs[0][r, pl.ds(cc * 16, 16)] = zero16
        return carry

    lax.fori_loop(0, CHUNK, zbody, 0)
    r0 = s * RPT
    for i in range(RPT // CHUNK):
        pltpu.sync_copy(bufs[0], acc.at[pl.ds(r0 + i * CHUNK, CHUNK)])
    plsc.subcore_barrier()

    def fire_gather(i, j):
        pltpu.async_copy(h_hbm.at[_chunk(src_v, j)], bufs[i], gsems[i])

    def wait_gather(i, j):
        pltpu.make_async_copy(h_hbm.at[_chunk(src_v, j)], bufs[i],
                              gsems[i]).wait()

    for i in range(NBUF):
        fire_gather(i, i)

    def body(k, carry):
        j0 = NBUF * k
        descs = []
        for i in range(NBUF):
            wait_gather(i, j0 + i)
            descs.append(
                pltpu.async_copy(bufs[i], acc.at[_chunk(dst_v, j0 + i)],
                                 ssems[i], add=True))
        for i in range(NBUF):
            descs[i].wait()
            jn = j0 + NBUF + i

            @pl.when(jn < NFULL)
            def _():
                fire_gather(i, jn)

        return carry

    lax.fori_loop(0, NFULL // NBUF, body, 0)
    pltpu.async_copy(h_hbm.at[_tail(src_v)], bufs[2].at[pl.ds(0, TAIL)],
                     gsems[2])
    d = []
    for i in range(NFULL - NBUF * (NFULL // NBUF)):
        j = NBUF * (NFULL // NBUF) + i
        wait_gather(i, j)
        d.append(pltpu.async_copy(bufs[i], acc.at[_chunk(dst_v, j)],
                                  ssems[i], add=True))
    pltpu.make_async_copy(h_hbm.at[_tail(src_v)], bufs[2].at[pl.ds(0, TAIL)],
                          gsems[2]).wait()
    d.append(pltpu.async_copy(bufs[2].at[pl.ds(0, TAIL)],
                              acc.at[_tail(dst_v)], ssems[2], add=True))
    for desc in d:
        desc.wait()

    plsc.subcore_barrier()

    @pl.when(c == 0)
    def _():
        pltpu.sync_copy(acc.at[pl.ds(r0, RPT)],
                        out_hbm.at[pl.ds(r0, RPT), pl.ds(0, HID)])

    @pl.when(c == 1)
    def _():
        pltpu.sync_copy(acc.at[pl.ds(r0, RPT)],
                        out_hbm.at[pl.ds(r0, RPT), pl.ds(HID, HID)])


def _tc_mm_body(x_ref, w_ref, mm_ref):
    mm_ref[...] = jnp.dot(x_ref[...], w_ref[...],
                          preferred_element_type=jnp.float32)


def _dis_matrix(disw, rows):
    d3 = lax.broadcast_in_dim(disw, (rows // 128, HID, 128), (0, 2))
    return jnp.swapaxes(d3, 1, 2).reshape(rows, HID)


def _tc1_body(mm_ref, d0_ref, d1_ref, dis_ref, hs_ref):
    disw = lax.rsqrt(1.0 + d0_ref[...] + d1_ref[...])
    dis_ref[...] = disw
    dmat = _dis_matrix(disw, N_PAD)
    hs_ref[pl.ds(0, N_NODES), :] = mm_ref[...] * dmat[:N_NODES, :]
    hs_ref[pl.ds(N_NODES, N_PAD - N_NODES), :] = jnp.zeros(
        (N_PAD - N_NODES, HID), jnp.float32)


BLK = 2048


def _tc2_body(p_ref, hs1_ref, dis_ref, b1_ref, w2_ref, hs2_ref):
    dmat = _dis_matrix(dis_ref[...], BLK)
    p = p_ref[...]
    h1 = jnp.maximum(
        dmat * (p[:, :HID] + p[:, HID:] + hs1_ref[...]) + b1_ref[...], 0.0
    )
    hs2_ref[...] = (
        jnp.dot(h1, w2_ref[...], preferred_element_type=jnp.float32) * dmat
    )


def _tc3_body(q_ref, hs2_ref, dis_ref, b2_ref, out_ref):
    dmat = _dis_matrix(dis_ref[...], BLK)
    q = q_ref[...]
    out_ref[...] = (
        dmat * (q[:, :HID] + q[:, HID:] + hs2_ref[...])
        + b2_ref[...]
    )


def kernel(x, edge_index, W1, b1, W2, b2):
    x = x.astype(jnp.float32)
    ei = edge_index.astype(jnp.int32)
    src1 = ei[0]
    dst1 = ei[1]

    degp = _deg_kernel(dst1)
    d0 = degp[0].reshape(DW, 128)
    d1 = degp[1].reshape(DW, 128)

    mm = pl.pallas_call(
        _tc_mm_body,
        out_shape=jax.ShapeDtypeStruct((N_NODES, HID), jnp.float32),
    )(x, W1)

    dis, hs1 = pl.pallas_call(
        _tc1_body,
        out_shape=[
            jax.ShapeDtypeStruct((DW, 128), jnp.float32),
            jax.ShapeDtypeStruct((N_PAD, HID), jnp.float32),
        ],
    )(mm, d0, d1)

    p = _agg_kernel(hs1, src1, dst1)

    hs2 = pl.pallas_call(
        _tc2_body,
        grid=(N_PAD // BLK,),
        in_specs=[
            pl.BlockSpec((BLK, 2 * HID), lambda k: (k, 0)),
            pl.BlockSpec((BLK, HID), lambda k: (k, 0)),
            pl.BlockSpec((BLK // 128, 128), lambda k: (k, 0)),
            pl.BlockSpec((1, HID), lambda k: (0, 0)),
            pl.BlockSpec((HID, HID), lambda k: (0, 0)),
        ],
        out_specs=pl.BlockSpec((BLK, HID), lambda k: (k, 0)),
        out_shape=jax.ShapeDtypeStruct((N_PAD, HID), jnp.float32),
    )(p, hs1, dis, b1.reshape(1, HID), W2)

    q = _agg_kernel(hs2, src1, dst1)

    out = pl.pallas_call(
        _tc3_body,
        grid=(N_PAD // BLK,),
        in_specs=[
            pl.BlockSpec((BLK, 2 * HID), lambda k: (k, 0)),
            pl.BlockSpec((BLK, HID), lambda k: (k, 0)),
            pl.BlockSpec((BLK // 128, 128), lambda k: (k, 0)),
            pl.BlockSpec((1, HID), lambda k: (0, 0)),
        ],
        out_specs=pl.BlockSpec((BLK, HID), lambda k: (k, 0)),
        out_shape=jax.ShapeDtypeStruct((N_NODES, HID), jnp.float32),
    )(q, hs2, dis, b2.reshape(1, HID))
    return out
